# baseline (speedup 1.0000x reference)
import jax
import jax.numpy as jnp
import numpy as np
from jax.experimental import pallas as pl


def _pos_kernel(col_ref, row_ref, out_ref):
    c = col_ref[...]
    r = row_ref[...]
    col_part = jnp.broadcast_to(c[None, :, :], (32, 32, 384)).reshape(1024, 384)
    row_part = jnp.broadcast_to(r[:, None, :], (32, 32, 384)).reshape(1024, 384)
    block = jnp.concatenate([col_part, row_part], axis=-1)
    out_ref[0] = block


def kernel(x, row_embed, col_embed):
    B = x.shape[0]
    h = w = int(np.sqrt(x.shape[1]))
    F = row_embed.shape[1]
    c32 = col_embed[:w]
    r32 = row_embed[:h]
    out = pl.pallas_call(
        _pos_kernel,
        grid=(B,),
        in_specs=[
            pl.BlockSpec((w, F), lambda b: (0, 0)),
            pl.BlockSpec((h, F), lambda b: (0, 0)),
        ],
        out_specs=pl.BlockSpec((1, h * w, 2 * F), lambda b: (b, 0, 0)),
        out_shape=jax.ShapeDtypeStruct((B, h * w, 2 * F), jnp.float32),
    )(c32, r32)
    return out
